# baseline (speedup 1.0000x reference)
import functools

import jax
import jax.numpy as jnp
from jax import lax
from jax.experimental import pallas as pl
from jax.experimental.pallas import tpu as pltpu
from jax.experimental.pallas import tpu_sc as plsc

_C = 100000
_B = 1024
_IGNORE = 0
_SMOOTHING = 0.1
_CONF = 1.0 - _SMOOTHING
_EPS = _SMOOTHING / (_C - 1)

_NC = 2
_NS = 16
_L = 16
_NW = _NC * _NS
_BPW = _B // _NW

_ROWS_SC = 512
_ROWS_TC = _B - _ROWS_SC
_DRW = _ROWS_SC // _NW

_W = 23 * 128
_NCHUNK = 34
_NVEC = _W // _L
_NVEC_LAST = (_C - (_NCHUNK - 1) * _W) // _L

_BR = 16
_TCG = _ROWS_TC // (2 * _BR)


def _sc_body(logit_hbm, tgt, y_out, s_out, tgt_v, tiles_v, wtgt_v, dbuf,
             val_v, sacc_v, sem_g, sem_d):
    wid = lax.axis_index("s") * _NC + lax.axis_index("c")
    base = wid * _BPW
    lanes = lax.iota(jnp.int32, _L)

    pltpu.sync_copy(tgt.at[pl.ds(base, _BPW)], tgt_v)
    tvecs = [tgt_v[pl.ds(k * _L, _L)] for k in range(_BPW // _L)]
    tscal = []
    for i in range(_BPW):
        t = jnp.sum(jnp.where(lanes == (i % _L), tvecs[i // _L], 0))
        tscal.append(t)
        row8 = base + (i // 8) * 8
        col128 = (t // 128) * 128
        pltpu.make_async_copy(
            logit_hbm.at[pl.ds(row8, 8), pl.ds(col128, 128)],
            tiles_v.at[i],
            sem_g,
        ).start()

    dr0 = _ROWS_TC + wid * _DRW
    pltpu.sync_copy(tgt.at[pl.ds(dr0, _DRW)], wtgt_v)
    wfull = jnp.where(wtgt_v[...] != _IGNORE, 1.0, 0.0)
    acc = jnp.zeros((_L,), jnp.float32)
    for tr in range(_DRW // 8):
        row8 = dr0 + 8 * tr
        wvs = []
        for r in range(8):
            w_r = jnp.sum(jnp.where(lanes == (tr * 8 + r), wfull, 0.0))
            wvs.append(jnp.full((_L,), w_r, jnp.float32))
        pltpu.make_async_copy(
            logit_hbm.at[pl.ds(row8, 8), pl.ds(0, _W)], dbuf.at[0], sem_d
        ).start()

        def chunk_body(k, a, row8=row8, wvs=wvs):
            slot = lax.rem(k, 2)
            nslot = lax.rem(k + 1, 2)

            @pl.when(k < _NCHUNK - 1)
            def _():
                pltpu.make_async_copy(
                    logit_hbm.at[pl.ds(row8, 8), pl.ds((k + 1) * _W, _W)],
                    dbuf.at[nslot],
                    sem_d,
                ).start()

            pltpu.make_async_copy(
                logit_hbm.at[pl.ds(row8, 8), pl.ds(0, _W)], dbuf.at[slot],
                sem_d,
            ).wait()
            nv = jnp.where(k == _NCHUNK - 1, _NVEC_LAST, _NVEC)

            def vbody(v, a2):
                for r in range(8):
                    a2 = a2 + dbuf[slot, r, pl.ds(v * _L, _L)] * wvs[r]
                return a2

            return lax.fori_loop(0, nv, vbody, a)

        acc = lax.fori_loop(0, _NCHUNK, chunk_body, acc)
    sacc_v[...] = acc
    pltpu.sync_copy(sacc_v, s_out.at[pl.ds(wid * _L, _L)])

    for i in range(_BPW):
        pltpu.make_async_copy(
            logit_hbm.at[pl.ds(0, 8), pl.ds(0, 128)], tiles_v.at[i], sem_g
        ).wait()
    for k in range(_BPW // _L):
        yacc = jnp.zeros((_L,), jnp.float32)
        for j in range(_L):
            i = k * _L + j
            t = tscal[i]
            sub = (base + i) % 8
            l16 = ((t % 128) // 16) * 16
            vec = tiles_v[i, sub, pl.ds(l16, 16)]
            y = jnp.sum(jnp.where(lanes == (t % 16), vec, 0.0))
            y = jnp.where(t != _IGNORE, y, 0.0)
            yacc = jnp.where(lanes == j, y, yacc)
        val_v[pl.ds(k * _L, _L)] = yacc
    pltpu.sync_copy(val_v, y_out.at[pl.ds(base, _BPW)])


@functools.lru_cache(maxsize=1)
def _sc_kernel():
    return pl.kernel(
        _sc_body,
        mesh=plsc.VectorSubcoreMesh(core_axis_name="c", subcore_axis_name="s"),
        compiler_params=pltpu.CompilerParams(needs_layout_passes=False),
        out_type=(
            jax.ShapeDtypeStruct((_B,), jnp.float32),
            jax.ShapeDtypeStruct((_NW * _L,), jnp.float32),
        ),
        scratch_types=[
            pltpu.VMEM((_BPW,), jnp.int32),
            pltpu.VMEM((_BPW, 8, 128), jnp.float32),
            pltpu.VMEM((_DRW,), jnp.int32),
            pltpu.VMEM((2, 8, _W), jnp.float32),
            pltpu.VMEM((_BPW,), jnp.float32),
            pltpu.VMEM((_L,), jnp.float32),
            pltpu.SemaphoreType.DMA,
            pltpu.SemaphoreType.DMA,
        ],
    )


def _tc_reduce_body(tgt1_ref, tgt2_ref, x1_ref, x2_ref, o_ref):
    j = pl.program_id(0)

    @pl.when(j == 0)
    def _():
        o_ref[0, 0] = 0.0

    w1 = (tgt1_ref[...] != _IGNORE).astype(jnp.float32)
    w2 = (tgt2_ref[...] != _IGNORE).astype(jnp.float32)
    o_ref[0, 0] += _EPS * (jnp.sum(x1_ref[...] * w1) + jnp.sum(x2_ref[...] * w2))


def _combine_body(stc_ref, y_ref, ssc_ref, o_ref):
    o_ref[0, 0] = -(
        stc_ref[0, 0]
        + _EPS * jnp.sum(ssc_ref[...])
        + (_CONF - _EPS) * jnp.sum(y_ref[...])
    )


def kernel(logit, target):
    y, s_sc = _sc_kernel()(logit, target)
    tgt2d = target.reshape(_B, 1)
    s_tc = pl.pallas_call(
        _tc_reduce_body,
        grid=(_TCG,),
        in_specs=[
            pl.BlockSpec((_BR, 1), lambda j: (j, 0)),
            pl.BlockSpec((_BR, 1), lambda j: (j + _TCG, 0)),
            pl.BlockSpec((_BR, _C), lambda j: (j, 0)),
            pl.BlockSpec((_BR, _C), lambda j: (j + _TCG, 0)),
        ],
        out_specs=pl.BlockSpec(memory_space=pltpu.SMEM),
        out_shape=jax.ShapeDtypeStruct((1, 1), jnp.float32),
    )(tgt2d, tgt2d, logit, logit)
    out = pl.pallas_call(
        _combine_body,
        in_specs=[
            pl.BlockSpec(memory_space=pltpu.SMEM),
            pl.BlockSpec((8, 128), lambda: (0, 0)),
            pl.BlockSpec((4, 128), lambda: (0, 0)),
        ],
        out_specs=pl.BlockSpec(memory_space=pltpu.SMEM),
        out_shape=jax.ShapeDtypeStruct((1, 1), jnp.float32),
    )(s_tc, y.reshape(8, 128), s_sc.reshape(4, 128))
    return out[0, 0]
